# baseline (speedup 1.0000x reference)
import functools

import jax
import jax.numpy as jnp
from jax import lax
from jax.experimental import pallas as pl
from jax.experimental.pallas import tpu as pltpu
from jax.experimental.pallas import tpu_sc as plsc

N = 10000
E = 320000
H = 32
DFEAT = 128
DEDGE = 16
DGLOB = 32
DN = H + DFEAT
DE = H + DEDGE
DU = H + DGLOB

F32 = jnp.float32

NC = 2
NS = 16
NW = NC * NS
C = 80
EPW = E // NW
KW = EPW // C
NPAD = 10240
RZ = NPAD // NS

_MESH = plsc.VectorSubcoreMesh(core_axis_name="c", subcore_axis_name="s")


def _tca_body(hnf_ref, nf_ref, wsh, wsn, wdh, wdn, p_ref, q_ref):
    hnf = hnf_ref[...]
    nf = nf_ref[...]
    p_ref[...] = (jnp.dot(hnf, wsh[...], preferred_element_type=F32)
                  + jnp.dot(nf, wsn[...], preferred_element_type=F32))
    q_ref[...] = (jnp.dot(hnf, wdh[...], preferred_element_type=F32)
                  + jnp.dot(nf, wdn[...], preferred_element_type=F32))


NBUF = 5
KSTEP = KW // NBUF

@functools.partial(
    pl.kernel,
    out_type=jax.ShapeDtypeStruct((E, DE), F32),
    mesh=_MESH,
    scratch_types=[
        pltpu.VMEM((EPW,), jnp.int32),
        pltpu.VMEM((EPW,), jnp.int32),
        pltpu.VMEM((NBUF, C, DE), F32),
        pltpu.VMEM((NBUF, C, DE), F32),
        pltpu.VMEM((NBUF, C, DE), F32),
        pltpu.SemaphoreType.DMA((NBUF,)),
        pltpu.SemaphoreType.DMA((NBUF,)),
        pltpu.SemaphoreType.DMA((NBUF,)),
    ],
    compiler_params=pltpu.CompilerParams(use_tc_tiling_on_sc=False),
)
def _sc_gather(p_hbm, q_hbm, src_hbm, dst_hbm, hefp_hbm, efp_hbm, g_hbm,
               idx_s, idx_d, bufp, bufq, gout, semp, semq, semw):
    del hefp_hbm, efp_hbm
    cid = lax.axis_index("c")
    sid = lax.axis_index("s")
    w = sid * NC + cid
    base = w * EPW
    pltpu.sync_copy(src_hbm.at[pl.ds(base, EPW)], idx_s)
    pltpu.sync_copy(dst_hbm.at[pl.ds(base, EPW)], idx_d)

    def gathers(k, ph):
        cp = pltpu.async_copy(p_hbm.at[idx_s.at[pl.ds(k * C, C)]],
                              bufp.at[ph], semp.at[ph])
        cq = pltpu.async_copy(q_hbm.at[idx_d.at[pl.ds(k * C, C)]],
                              bufq.at[ph], semq.at[ph])
        return cp, cq

    for ph in range(NBUF):
        gathers(ph, ph)

    def body(j, carry):
        for ph in range(NBUF):
            k = j * NBUF + ph
            pltpu.make_async_copy(p_hbm.at[idx_s.at[pl.ds(0, C)]],
                                  bufp.at[ph], semp.at[ph]).wait()
            pltpu.make_async_copy(q_hbm.at[idx_d.at[pl.ds(0, C)]],
                                  bufq.at[ph], semq.at[ph]).wait()
            @pl.when(j > 0)
            def _():
                pltpu.make_async_copy(gout.at[ph], g_hbm.at[pl.ds(0, C)],
                                      semw.at[ph]).wait()
            def addrow(r, c2):
                for cc in range(DE // 16):
                    gout[ph, r, pl.ds(cc * 16, 16)] = (
                        bufp[ph, r, pl.ds(cc * 16, 16)]
                        + bufq[ph, r, pl.ds(cc * 16, 16)])
                return c2
            lax.fori_loop(0, C, addrow, 0)
            pltpu.async_copy(gout.at[ph], g_hbm.at[pl.ds(base + k * C, C)],
                             semw.at[ph])
            kn = jnp.minimum(k + NBUF, KW - 1)
            @pl.when(k + NBUF < KW)
            def _():
                gathers(kn, ph)
        return carry

    lax.fori_loop(0, KSTEP, body, 0)
    for ph in range(NBUF):
        pltpu.make_async_copy(gout.at[ph], g_hbm.at[pl.ds(0, C)],
                              semw.at[ph]).wait()


EB = 16000
PK = 8
EBP = EB // PK

def _tcc_body(g_ref, hefp_ref, efp_ref, w1bd, w2bd, whu8, wu8, hu_ref, u_ref,
              b8_ref, o32_ref, o16_ref):
    ec8 = (jnp.dot(hu_ref[...], whu8[...], preferred_element_type=F32)
           + jnp.dot(u_ref[...], wu8[...], preferred_element_type=F32)
           + b8_ref[...])
    g8 = jnp.reshape(g_ref[...], (EBP, PK * DE))
    hef8 = jnp.reshape(hefp_ref[...], (EBP, PK * H))
    s = (g8
         + jnp.dot(hef8, w1bd[...], preferred_element_type=F32)
         + jnp.dot(efp_ref[...], w2bd[...], preferred_element_type=F32)
         + ec8)
    t = jnp.tanh(s)
    h32 = jnp.concatenate([t[:, DE * j:DE * j + H] for j in range(PK)], axis=1)
    o32_ref[...] = jnp.reshape(h32, (EB * H // 128, 128))
    h16 = jnp.concatenate([t[:, DE * j + H:DE * (j + 1)] for j in range(PK)],
                          axis=1)
    o16_ref[...] = jnp.reshape(h16, (EB * DEDGE // 128, 128))


FZ = RZ // 2

@functools.partial(
    pl.kernel,
    out_type=(jax.ShapeDtypeStruct((NC, NPAD, 128), F32),
              jax.ShapeDtypeStruct((E, H), F32),
              jax.ShapeDtypeStruct((E, DEDGE), F32)),
    mesh=_MESH,
    scratch_types=[
        pltpu.VMEM((KW, C), jnp.int32),
        pltpu.VMEM((NBUF, C, H), F32),
        pltpu.VMEM((NBUF, C, DEDGE), F32),
        pltpu.VMEM_SHARED((NPAD, H), F32),
        pltpu.VMEM_SHARED((NPAD, DEDGE), F32),
        pltpu.VMEM((FZ, H), F32),
        pltpu.VMEM((FZ, DEDGE), F32),
        pltpu.VMEM((FZ, 128), F32),
        pltpu.SemaphoreType.DMA((NBUF,)),
        pltpu.SemaphoreType.DMA((NBUF,)),
        pltpu.SemaphoreType.DMA((NBUF,)),
        pltpu.SemaphoreType.DMA((NBUF,)),
        pltpu.SemaphoreType.DMA,
        pltpu.SemaphoreType.DMA,
    ],
    compiler_params=pltpu.CompilerParams(use_tc_tiling_on_sc=False),
)
def _sc_scatter(h32_hbm, h16_hbm, dst3_hbm, a48_hbm, h32c_hbm, h16c_hbm,
                idx2, buf32, buf16, sh32, sh16, f32b, f16b, mout,
                sem32, sem16, semc32, semc16, ssc32, ssc16):
    cid = lax.axis_index("c")
    sid = lax.axis_index("s")
    w = sid * NC + cid
    base = w * EPW
    zv = jnp.zeros((16,), F32)

    def zrow(r, carry):
        f32b[r, pl.ds(0, 16)] = zv
        f32b[r, pl.ds(16, 16)] = zv
        f16b[r, pl.ds(0, 16)] = zv
        return carry

    lax.fori_loop(0, FZ, zrow, 0)
    for m in range(RZ // FZ):
        pltpu.sync_copy(f32b, sh32.at[pl.ds(sid * RZ + m * FZ, FZ)])
        pltpu.sync_copy(f16b, sh16.at[pl.ds(sid * RZ + m * FZ, FZ)])
    pltpu.sync_copy(dst3_hbm.at[w], idx2)
    plsc.subcore_barrier()

    def loads(k, ph):
        off = base + k * C
        pltpu.async_copy(h32_hbm.at[pl.ds(off, C)], buf32.at[ph], sem32.at[ph])
        pltpu.async_copy(h16_hbm.at[pl.ds(off, C)], buf16.at[ph], sem16.at[ph])

    for ph in range(NBUF):
        loads(ph, ph)

    def body(j, carry):
        for ph in range(NBUF):
            k = j * NBUF + ph
            off = base + k * C
            pltpu.make_async_copy(h32_hbm.at[pl.ds(0, C)], buf32.at[ph],
                                  sem32.at[ph]).wait()
            pltpu.make_async_copy(h16_hbm.at[pl.ds(0, C)], buf16.at[ph],
                                  sem16.at[ph]).wait()
            c32 = pltpu.async_copy(buf32.at[ph], h32c_hbm.at[pl.ds(off, C)],
                                   semc32.at[ph])
            c16 = pltpu.async_copy(buf16.at[ph], h16c_hbm.at[pl.ds(off, C)],
                                   semc16.at[ph])
            s32 = pltpu.async_copy(buf32.at[ph], sh32.at[idx2.at[k]], ssc32,
                                   add=True)
            s16 = pltpu.async_copy(buf16.at[ph], sh16.at[idx2.at[k]], ssc16,
                                   add=True)
            s32.wait()
            s16.wait()
            c32.wait()
            c16.wait()
            kn = jnp.minimum(k + NBUF, KW - 1)
            @pl.when(k + NBUF < KW)
            def _():
                loads(kn, ph)
        return carry

    lax.fori_loop(0, KSTEP, body, 0)
    plsc.subcore_barrier()
    for m in range(RZ // FZ):
        row0 = sid * RZ + m * FZ
        pltpu.sync_copy(sh32.at[pl.ds(row0, FZ)], f32b)
        pltpu.sync_copy(sh16.at[pl.ds(row0, FZ)], f16b)

        def mrow(r, carry):
            mout[r, pl.ds(0, 16)] = f32b[r, pl.ds(0, 16)]
            mout[r, pl.ds(16, 16)] = f32b[r, pl.ds(16, 16)]
            mout[r, pl.ds(32, 16)] = f16b[r, pl.ds(0, 16)]
            return carry

        lax.fori_loop(0, FZ, mrow, 0)
        pltpu.sync_copy(mout, a48_hbm.at[cid, pl.ds(row0, FZ)])


NB = 2000
NGRID = N // NB

def _tce_body(hnf_ref, nf_ref, a48_ref, wnh, wnn, wna32, wna16,
              wnhu, wnu, wnb, wghu, wgu, wgn, wge, wgb, hu_ref, u_ref,
              hnfn_ref, unfo_ref, hun_ref, uuo_ref, accn, acce):
    i = pl.program_id(0)
    a48 = a48_ref[0] + a48_ref[1]
    a32 = a48[:, :H]
    a16 = a48[:, H:DE]
    nc_term = (jnp.dot(hu_ref[...], wnhu[...], preferred_element_type=F32)
               + jnp.dot(u_ref[...], wnu[...], preferred_element_type=F32)
               + wnb[...])
    pre = (jnp.dot(hnf_ref[...], wnh[...], preferred_element_type=F32)
           + jnp.dot(nf_ref[...], wnn[...], preferred_element_type=F32)
           + jnp.dot(a32, wna32[...], preferred_element_type=F32)
           + jnp.dot(a16, wna16[...], preferred_element_type=F32)
           + nc_term)
    unf = jnp.tanh(pre)
    hnfn_ref[...] = unf[:, :H]
    unfo_ref[...] = unf[:, H:]

    @pl.when(i == 0)
    def _():
        accn[...] = jnp.zeros_like(accn)
        acce[...] = jnp.zeros_like(acce)

    accn[...] += jnp.sum(unf, axis=0, keepdims=True)
    acce[...] += jnp.concatenate(
        [jnp.sum(a32, axis=0, keepdims=True),
         jnp.sum(a16, axis=0, keepdims=True)], axis=1)

    ug = (jnp.dot(hu_ref[...], wghu[...], preferred_element_type=F32)
          + jnp.dot(u_ref[...], wgu[...], preferred_element_type=F32)
          + wgb[...])
    uu = jnp.tanh(ug + jnp.dot(accn[...], wgn[...], preferred_element_type=F32)
                  + jnp.dot(acce[...], wge[...], preferred_element_type=F32))
    hun_ref[...] = uu[:, :H]
    uuo_ref[...] = uu[:, H:]


def kernel(nf, ef, u, edge_index, hnf, hef, hu, We_w, We_b, Wn_w, Wn_b, Wg_w, Wg_b):
    src2 = edge_index[0]
    dst2 = edge_index[1]

    p, q = pl.pallas_call(
        _tca_body,
        out_shape=(jax.ShapeDtypeStruct((N, DE), F32),
                   jax.ShapeDtypeStruct((N, DE), F32)),
    )(hnf, nf, We_w[0:H], We_w[H:DN], We_w[DN:DN + H], We_w[DN + H:2 * DN])

    hefp = hef.reshape(E * H // 128, 128)
    efp = ef.reshape(E * DEDGE // 128, 128)
    g = _sc_gather(p, q, src2, dst2, hefp, efp)

    import jax.scipy.linalg as _jsl
    w1 = We_w[2 * DN:2 * DN + H]
    w2 = We_w[2 * DN + H:2 * DN + DE]
    w1bd = _jsl.block_diag(*([w1] * PK))
    w2bd = _jsl.block_diag(*([w2] * PK))
    whu8 = jnp.tile(We_w[2 * DN + DE:2 * DN + DE + H], (1, PK))
    wu8 = jnp.tile(We_w[2 * DN + DE + H:], (1, PK))
    b8 = jnp.tile(We_b.reshape(1, DE), (1, PK))
    g2 = g.reshape(E * DE // 128, 128)
    grid_e = E // EB
    h32p, h16p = pl.pallas_call(
        _tcc_body,
        grid=(grid_e,),
        in_specs=[
            pl.BlockSpec((EB * DE // 128, 128), lambda i: (i, 0)),
            pl.BlockSpec((EB * H // 128, 128), lambda i: (i, 0)),
            pl.BlockSpec((EB * DEDGE // 128, 128), lambda i: (i, 0)),
            pl.BlockSpec((PK * H, PK * DE), lambda i: (0, 0)),
            pl.BlockSpec((PK * DEDGE, PK * DE), lambda i: (0, 0)),
            pl.BlockSpec((H, PK * DE), lambda i: (0, 0)),
            pl.BlockSpec((DGLOB, PK * DE), lambda i: (0, 0)),
            pl.BlockSpec((1, H), lambda i: (0, 0)),
            pl.BlockSpec((1, DGLOB), lambda i: (0, 0)),
            pl.BlockSpec((1, PK * DE), lambda i: (0, 0)),
        ],
        out_specs=(pl.BlockSpec((EB * H // 128, 128), lambda i: (i, 0)),
                   pl.BlockSpec((EB * DEDGE // 128, 128), lambda i: (i, 0))),
        out_shape=(jax.ShapeDtypeStruct((E * H // 128, 128), F32),
                   jax.ShapeDtypeStruct((E * DEDGE // 128, 128), F32)),
        compiler_params=pltpu.CompilerParams(
            dimension_semantics=("parallel",)),
    )(g2, hefp, efp, w1bd, w2bd, whu8, wu8, hu, u, b8)
    h32 = h32p.reshape(E, H)
    h16 = h16p.reshape(E, DEDGE)

    dst3 = dst2.reshape(NW, KW, C)
    a48, h32c, h16c = _sc_scatter(h32, h16, dst3)

    hnfn, unfo, hun, uuo = pl.pallas_call(
        _tce_body,
        grid=(NGRID,),
        in_specs=[
            pl.BlockSpec((NB, H), lambda i: (i, 0)),
            pl.BlockSpec((NB, DFEAT), lambda i: (i, 0)),
            pl.BlockSpec((NC, NB, 128), lambda i: (0, i, 0)),
            pl.BlockSpec((H, DN), lambda i: (0, 0)),
            pl.BlockSpec((DFEAT, DN), lambda i: (0, 0)),
            pl.BlockSpec((H, DN), lambda i: (0, 0)),
            pl.BlockSpec((DEDGE, DN), lambda i: (0, 0)),
            pl.BlockSpec((H, DN), lambda i: (0, 0)),
            pl.BlockSpec((DGLOB, DN), lambda i: (0, 0)),
            pl.BlockSpec((1, DN), lambda i: (0, 0)),
            pl.BlockSpec((H, DU), lambda i: (0, 0)),
            pl.BlockSpec((DGLOB, DU), lambda i: (0, 0)),
            pl.BlockSpec((DN, DU), lambda i: (0, 0)),
            pl.BlockSpec((DE, DU), lambda i: (0, 0)),
            pl.BlockSpec((1, DU), lambda i: (0, 0)),
            pl.BlockSpec((1, H), lambda i: (0, 0)),
            pl.BlockSpec((1, DGLOB), lambda i: (0, 0)),
        ],
        out_specs=(pl.BlockSpec((NB, H), lambda i: (i, 0)),
                   pl.BlockSpec((NB, DFEAT), lambda i: (i, 0)),
                   pl.BlockSpec((1, H), lambda i: (0, 0)),
                   pl.BlockSpec((1, H), lambda i: (0, 0))),
        out_shape=(jax.ShapeDtypeStruct((N, H), F32),
                   jax.ShapeDtypeStruct((N, DFEAT), F32),
                   jax.ShapeDtypeStruct((1, H), F32),
                   jax.ShapeDtypeStruct((1, H), F32)),
        scratch_shapes=[pltpu.VMEM((1, DN), F32), pltpu.VMEM((1, DE), F32)],
        compiler_params=pltpu.CompilerParams(
            dimension_semantics=("arbitrary",)),
    )(hnf, nf, a48,
      Wn_w[0:H], Wn_w[H:DN], Wn_w[DN:DN + H], Wn_w[DN + H:DN + DE],
      Wn_w[DN + DE:DN + DE + H], Wn_w[DN + DE + H:], Wn_b.reshape(1, DN),
      Wg_w[0:H], Wg_w[H:DU], Wg_w[DU:DU + DN], Wg_w[DU + DN:],
      Wg_b.reshape(1, DU), hu, u)

    return (hnfn, h32c, hun, unfo, h16c, uuo)


# speedup vs baseline: 1.2718x; 1.2718x over previous
import functools

import jax
import jax.numpy as jnp
from jax import lax
from jax.experimental import pallas as pl
from jax.experimental.pallas import tpu as pltpu
from jax.experimental.pallas import tpu_sc as plsc

N = 10000
E = 320000
H = 32
DFEAT = 128
DEDGE = 16
DGLOB = 32
DN = H + DFEAT
DE = H + DEDGE
DU = H + DGLOB

F32 = jnp.float32

NC = 2
NS = 16
NW = NC * NS
C = 80
EPW = E // NW
KW = EPW // C
NPAD = 10240
RZ = NPAD // NS

_MESH = plsc.VectorSubcoreMesh(core_axis_name="c", subcore_axis_name="s")


def _tca_body(hnf_ref, nf_ref, wsh, wsn, wdh, wdn, p_ref, q_ref):
    hnf = hnf_ref[...]
    nf = nf_ref[...]
    p_ref[...] = (jnp.dot(hnf, wsh[...], preferred_element_type=F32)
                  + jnp.dot(nf, wsn[...], preferred_element_type=F32))
    q_ref[...] = (jnp.dot(hnf, wdh[...], preferred_element_type=F32)
                  + jnp.dot(nf, wdn[...], preferred_element_type=F32))


NBUF = 5
KSTEP = KW // NBUF

@functools.partial(
    pl.kernel,
    out_type=jax.ShapeDtypeStruct((E, DE), F32),
    mesh=_MESH,
    scratch_types=[
        pltpu.VMEM((EPW,), jnp.int32),
        pltpu.VMEM((EPW,), jnp.int32),
        pltpu.VMEM((NBUF, C, DE), F32),
        pltpu.VMEM((NBUF, C, DE), F32),
        pltpu.VMEM((NBUF, C, DE), F32),
        pltpu.SemaphoreType.DMA((NBUF,)),
        pltpu.SemaphoreType.DMA((NBUF,)),
        pltpu.SemaphoreType.DMA((NBUF,)),
    ],
    compiler_params=pltpu.CompilerParams(use_tc_tiling_on_sc=False),
)
def _sc_gather(p_hbm, q_hbm, src_hbm, dst_hbm, g_hbm,
               idx_s, idx_d, bufp, bufq, gout, semp, semq, semw):
    cid = lax.axis_index("c")
    sid = lax.axis_index("s")
    w = sid * NC + cid
    base = w * EPW
    pltpu.sync_copy(src_hbm.at[pl.ds(base, EPW)], idx_s)
    pltpu.sync_copy(dst_hbm.at[pl.ds(base, EPW)], idx_d)

    def gathers(k, ph):
        cp = pltpu.async_copy(p_hbm.at[idx_s.at[pl.ds(k * C, C)]],
                              bufp.at[ph], semp.at[ph])
        cq = pltpu.async_copy(q_hbm.at[idx_d.at[pl.ds(k * C, C)]],
                              bufq.at[ph], semq.at[ph])
        return cp, cq

    for ph in range(NBUF):
        gathers(ph, ph)

    def body(j, carry):
        for ph in range(NBUF):
            k = j * NBUF + ph
            pltpu.make_async_copy(p_hbm.at[idx_s.at[pl.ds(0, C)]],
                                  bufp.at[ph], semp.at[ph]).wait()
            pltpu.make_async_copy(q_hbm.at[idx_d.at[pl.ds(0, C)]],
                                  bufq.at[ph], semq.at[ph]).wait()
            @pl.when(j > 0)
            def _():
                pltpu.make_async_copy(gout.at[ph], g_hbm.at[pl.ds(0, C)],
                                      semw.at[ph]).wait()
            def addrow(r, c2):
                for cc in range(DE // 16):
                    gout[ph, r, pl.ds(cc * 16, 16)] = (
                        bufp[ph, r, pl.ds(cc * 16, 16)]
                        + bufq[ph, r, pl.ds(cc * 16, 16)])
                return c2
            lax.fori_loop(0, C, addrow, 0)
            pltpu.async_copy(gout.at[ph], g_hbm.at[pl.ds(base + k * C, C)],
                             semw.at[ph])
            kn = jnp.minimum(k + NBUF, KW - 1)
            @pl.when(k + NBUF < KW)
            def _():
                gathers(kn, ph)
        return carry

    lax.fori_loop(0, KSTEP, body, 0)
    for ph in range(NBUF):
        pltpu.make_async_copy(gout.at[ph], g_hbm.at[pl.ds(0, C)],
                              semw.at[ph]).wait()


EB = 16000
PK = 8
EBP = EB // PK

def _tcc_body(g_ref, hefp_ref, efp_ref, w1bd, w2bd, whu8, wu8, hu_ref, u_ref,
              b8_ref, o32_ref, o16_ref):
    ec8 = (jnp.dot(hu_ref[...], whu8[...], preferred_element_type=F32)
           + jnp.dot(u_ref[...], wu8[...], preferred_element_type=F32)
           + b8_ref[...])
    g8 = jnp.reshape(g_ref[...], (EBP, PK * DE))
    hef8 = jnp.reshape(hefp_ref[...], (EBP, PK * H))
    s = (g8
         + jnp.dot(hef8, w1bd[...], preferred_element_type=F32)
         + jnp.dot(efp_ref[...], w2bd[...], preferred_element_type=F32)
         + ec8)
    t = jnp.tanh(s)
    h32 = jnp.concatenate([t[:, DE * j:DE * j + H] for j in range(PK)], axis=1)
    o32_ref[...] = jnp.reshape(h32, (EB * H // 128, 128))
    h16 = jnp.concatenate([t[:, DE * j + H:DE * (j + 1)] for j in range(PK)],
                          axis=1)
    o16_ref[...] = jnp.reshape(h16, (EB * DEDGE // 128, 128))


FZ = RZ // 2

@functools.partial(
    pl.kernel,
    out_type=(jax.ShapeDtypeStruct((NC, NPAD, 128), F32),
              jax.ShapeDtypeStruct((E, H), F32),
              jax.ShapeDtypeStruct((E, DEDGE), F32)),
    mesh=_MESH,
    scratch_types=[
        pltpu.VMEM((KW, C), jnp.int32),
        pltpu.VMEM((NBUF, C, H), F32),
        pltpu.VMEM((NBUF, C, DEDGE), F32),
        pltpu.VMEM_SHARED((NPAD, H), F32),
        pltpu.VMEM_SHARED((NPAD, DEDGE), F32),
        pltpu.VMEM((FZ, H), F32),
        pltpu.VMEM((FZ, DEDGE), F32),
        pltpu.VMEM((FZ, 128), F32),
        pltpu.SemaphoreType.DMA((NBUF,)),
        pltpu.SemaphoreType.DMA((NBUF,)),
        pltpu.SemaphoreType.DMA((NBUF,)),
        pltpu.SemaphoreType.DMA((NBUF,)),
        pltpu.SemaphoreType.DMA,
        pltpu.SemaphoreType.DMA,
    ],
    compiler_params=pltpu.CompilerParams(use_tc_tiling_on_sc=False),
)
def _sc_scatter(h32_hbm, h16_hbm, dst3_hbm, a48_hbm, h32c_hbm, h16c_hbm,
                idx2, buf32, buf16, sh32, sh16, f32b, f16b, mout,
                sem32, sem16, semc32, semc16, ssc32, ssc16):
    cid = lax.axis_index("c")
    sid = lax.axis_index("s")
    w = sid * NC + cid
    base = w * EPW
    zv = jnp.zeros((16,), F32)

    def zrow(r, carry):
        f32b[r, pl.ds(0, 16)] = zv
        f32b[r, pl.ds(16, 16)] = zv
        f16b[r, pl.ds(0, 16)] = zv
        return carry

    lax.fori_loop(0, FZ, zrow, 0)
    for m in range(RZ // FZ):
        pltpu.sync_copy(f32b, sh32.at[pl.ds(sid * RZ + m * FZ, FZ)])
        pltpu.sync_copy(f16b, sh16.at[pl.ds(sid * RZ + m * FZ, FZ)])
    pltpu.sync_copy(dst3_hbm.at[w], idx2)
    plsc.subcore_barrier()

    def loads(k, ph):
        off = base + k * C
        pltpu.async_copy(h32_hbm.at[pl.ds(off, C)], buf32.at[ph], sem32.at[ph])
        pltpu.async_copy(h16_hbm.at[pl.ds(off, C)], buf16.at[ph], sem16.at[ph])

    for ph in range(NBUF):
        loads(ph, ph)

    def body(j, carry):
        for ph in range(NBUF):
            k = j * NBUF + ph
            off = base + k * C
            pltpu.make_async_copy(h32_hbm.at[pl.ds(0, C)], buf32.at[ph],
                                  sem32.at[ph]).wait()
            pltpu.make_async_copy(h16_hbm.at[pl.ds(0, C)], buf16.at[ph],
                                  sem16.at[ph]).wait()
            c32 = pltpu.async_copy(buf32.at[ph], h32c_hbm.at[pl.ds(off, C)],
                                   semc32.at[ph])
            c16 = pltpu.async_copy(buf16.at[ph], h16c_hbm.at[pl.ds(off, C)],
                                   semc16.at[ph])
            s32 = pltpu.async_copy(buf32.at[ph], sh32.at[idx2.at[k]], ssc32,
                                   add=True)
            s16 = pltpu.async_copy(buf16.at[ph], sh16.at[idx2.at[k]], ssc16,
                                   add=True)
            s32.wait()
            s16.wait()
            c32.wait()
            c16.wait()
            kn = jnp.minimum(k + NBUF, KW - 1)
            @pl.when(k + NBUF < KW)
            def _():
                loads(kn, ph)
        return carry

    lax.fori_loop(0, KSTEP, body, 0)
    plsc.subcore_barrier()
    for m in range(RZ // FZ):
        row0 = sid * RZ + m * FZ
        pltpu.sync_copy(sh32.at[pl.ds(row0, FZ)], f32b)
        pltpu.sync_copy(sh16.at[pl.ds(row0, FZ)], f16b)

        def mrow(r, carry):
            mout[r, pl.ds(0, 16)] = f32b[r, pl.ds(0, 16)]
            mout[r, pl.ds(16, 16)] = f32b[r, pl.ds(16, 16)]
            mout[r, pl.ds(32, 16)] = f16b[r, pl.ds(0, 16)]
            return carry

        lax.fori_loop(0, FZ, mrow, 0)
        pltpu.sync_copy(mout, a48_hbm.at[cid, pl.ds(row0, FZ)])


NB = 2000
NGRID = N // NB

def _tce_body(hnf_ref, nf_ref, a48_ref, wnh, wnn, wna32, wna16,
              wnhu, wnu, wnb, wghu, wgu, wgn, wge, wgb, hu_ref, u_ref,
              hnfn_ref, unfo_ref, hun_ref, uuo_ref, accn, acce):
    i = pl.program_id(0)
    a48 = a48_ref[0] + a48_ref[1]
    a32 = a48[:, :H]
    a16 = a48[:, H:DE]
    nc_term = (jnp.dot(hu_ref[...], wnhu[...], preferred_element_type=F32)
               + jnp.dot(u_ref[...], wnu[...], preferred_element_type=F32)
               + wnb[...])
    pre = (jnp.dot(hnf_ref[...], wnh[...], preferred_element_type=F32)
           + jnp.dot(nf_ref[...], wnn[...], preferred_element_type=F32)
           + jnp.dot(a32, wna32[...], preferred_element_type=F32)
           + jnp.dot(a16, wna16[...], preferred_element_type=F32)
           + nc_term)
    unf = jnp.tanh(pre)
    hnfn_ref[...] = unf[:, :H]
    unfo_ref[...] = unf[:, H:]

    @pl.when(i == 0)
    def _():
        accn[...] = jnp.zeros_like(accn)
        acce[...] = jnp.zeros_like(acce)

    accn[...] += jnp.sum(unf, axis=0, keepdims=True)
    acce[...] += jnp.concatenate(
        [jnp.sum(a32, axis=0, keepdims=True),
         jnp.sum(a16, axis=0, keepdims=True)], axis=1)

    ug = (jnp.dot(hu_ref[...], wghu[...], preferred_element_type=F32)
          + jnp.dot(u_ref[...], wgu[...], preferred_element_type=F32)
          + wgb[...])
    uu = jnp.tanh(ug + jnp.dot(accn[...], wgn[...], preferred_element_type=F32)
                  + jnp.dot(acce[...], wge[...], preferred_element_type=F32))
    hun_ref[...] = uu[:, :H]
    uuo_ref[...] = uu[:, H:]


def kernel(nf, ef, u, edge_index, hnf, hef, hu, We_w, We_b, Wn_w, Wn_b, Wg_w, Wg_b):
    src2 = edge_index[0]
    dst2 = edge_index[1]

    p, q = pl.pallas_call(
        _tca_body,
        out_shape=(jax.ShapeDtypeStruct((N, DE), F32),
                   jax.ShapeDtypeStruct((N, DE), F32)),
    )(hnf, nf, We_w[0:H], We_w[H:DN], We_w[DN:DN + H], We_w[DN + H:2 * DN])

    hefp = hef.reshape(E * H // 128, 128)
    efp = ef.reshape(E * DEDGE // 128, 128)
    g = _sc_gather(p, q, src2, dst2)

    import jax.scipy.linalg as _jsl
    w1 = We_w[2 * DN:2 * DN + H]
    w2 = We_w[2 * DN + H:2 * DN + DE]
    w1bd = _jsl.block_diag(*([w1] * PK))
    w2bd = _jsl.block_diag(*([w2] * PK))
    whu8 = jnp.tile(We_w[2 * DN + DE:2 * DN + DE + H], (1, PK))
    wu8 = jnp.tile(We_w[2 * DN + DE + H:], (1, PK))
    b8 = jnp.tile(We_b.reshape(1, DE), (1, PK))
    g2 = g.reshape(E * DE // 128, 128)
    grid_e = E // EB
    h32p, h16p = pl.pallas_call(
        _tcc_body,
        grid=(grid_e,),
        in_specs=[
            pl.BlockSpec((EB * DE // 128, 128), lambda i: (i, 0)),
            pl.BlockSpec((EB * H // 128, 128), lambda i: (i, 0)),
            pl.BlockSpec((EB * DEDGE // 128, 128), lambda i: (i, 0)),
            pl.BlockSpec((PK * H, PK * DE), lambda i: (0, 0)),
            pl.BlockSpec((PK * DEDGE, PK * DE), lambda i: (0, 0)),
            pl.BlockSpec((H, PK * DE), lambda i: (0, 0)),
            pl.BlockSpec((DGLOB, PK * DE), lambda i: (0, 0)),
            pl.BlockSpec((1, H), lambda i: (0, 0)),
            pl.BlockSpec((1, DGLOB), lambda i: (0, 0)),
            pl.BlockSpec((1, PK * DE), lambda i: (0, 0)),
        ],
        out_specs=(pl.BlockSpec((EB * H // 128, 128), lambda i: (i, 0)),
                   pl.BlockSpec((EB * DEDGE // 128, 128), lambda i: (i, 0))),
        out_shape=(jax.ShapeDtypeStruct((E * H // 128, 128), F32),
                   jax.ShapeDtypeStruct((E * DEDGE // 128, 128), F32)),
        compiler_params=pltpu.CompilerParams(
            dimension_semantics=("parallel",)),
    )(g2, hefp, efp, w1bd, w2bd, whu8, wu8, hu, u, b8)
    h32 = h32p.reshape(E, H)
    h16 = h16p.reshape(E, DEDGE)

    dst3 = dst2.reshape(NW, KW, C)
    a48, h32c, h16c = _sc_scatter(h32, h16, dst3)

    hnfn, unfo, hun, uuo = pl.pallas_call(
        _tce_body,
        grid=(NGRID,),
        in_specs=[
            pl.BlockSpec((NB, H), lambda i: (i, 0)),
            pl.BlockSpec((NB, DFEAT), lambda i: (i, 0)),
            pl.BlockSpec((NC, NB, 128), lambda i: (0, i, 0)),
            pl.BlockSpec((H, DN), lambda i: (0, 0)),
            pl.BlockSpec((DFEAT, DN), lambda i: (0, 0)),
            pl.BlockSpec((H, DN), lambda i: (0, 0)),
            pl.BlockSpec((DEDGE, DN), lambda i: (0, 0)),
            pl.BlockSpec((H, DN), lambda i: (0, 0)),
            pl.BlockSpec((DGLOB, DN), lambda i: (0, 0)),
            pl.BlockSpec((1, DN), lambda i: (0, 0)),
            pl.BlockSpec((H, DU), lambda i: (0, 0)),
            pl.BlockSpec((DGLOB, DU), lambda i: (0, 0)),
            pl.BlockSpec((DN, DU), lambda i: (0, 0)),
            pl.BlockSpec((DE, DU), lambda i: (0, 0)),
            pl.BlockSpec((1, DU), lambda i: (0, 0)),
            pl.BlockSpec((1, H), lambda i: (0, 0)),
            pl.BlockSpec((1, DGLOB), lambda i: (0, 0)),
        ],
        out_specs=(pl.BlockSpec((NB, H), lambda i: (i, 0)),
                   pl.BlockSpec((NB, DFEAT), lambda i: (i, 0)),
                   pl.BlockSpec((1, H), lambda i: (0, 0)),
                   pl.BlockSpec((1, H), lambda i: (0, 0))),
        out_shape=(jax.ShapeDtypeStruct((N, H), F32),
                   jax.ShapeDtypeStruct((N, DFEAT), F32),
                   jax.ShapeDtypeStruct((1, H), F32),
                   jax.ShapeDtypeStruct((1, H), F32)),
        scratch_shapes=[pltpu.VMEM((1, DN), F32), pltpu.VMEM((1, DE), F32)],
        compiler_params=pltpu.CompilerParams(
            dimension_semantics=("arbitrary",)),
    )(hnf, nf, a48,
      Wn_w[0:H], Wn_w[H:DN], Wn_w[DN:DN + H], Wn_w[DN + H:DN + DE],
      Wn_w[DN + DE:DN + DE + H], Wn_w[DN + DE + H:], Wn_b.reshape(1, DN),
      Wg_w[0:H], Wg_w[H:DU], Wg_w[DU:DU + DN], Wg_w[DU + DN:],
      Wg_b.reshape(1, DU), hu, u)

    return (hnfn, h32c, hun, unfo, h16c, uuo)


# speedup vs baseline: 1.3811x; 1.0859x over previous
import functools

import jax
import jax.numpy as jnp
from jax import lax
from jax.experimental import pallas as pl
from jax.experimental.pallas import tpu as pltpu
from jax.experimental.pallas import tpu_sc as plsc

N = 10000
E = 320000
H = 32
DFEAT = 128
DEDGE = 16
DGLOB = 32
DN = H + DFEAT
DE = H + DEDGE
DU = H + DGLOB

F32 = jnp.float32

NC = 2
NS = 16
NW = NC * NS
C = 80
EPW = E // NW
KW = EPW // C
NPAD = 10240
RZ = NPAD // NS

_MESH = plsc.VectorSubcoreMesh(core_axis_name="c", subcore_axis_name="s")


def _tca_body(hnf_ref, nf_ref, wsh, wsn, wdh, wdn, p_ref, q_ref):
    hnf = hnf_ref[...]
    nf = nf_ref[...]
    p_ref[...] = (jnp.dot(hnf, wsh[...], preferred_element_type=F32)
                  + jnp.dot(nf, wsn[...], preferred_element_type=F32))
    q_ref[...] = (jnp.dot(hnf, wdh[...], preferred_element_type=F32)
                  + jnp.dot(nf, wdn[...], preferred_element_type=F32))


NBUF = 5
KSTEP = KW // NBUF

@functools.partial(
    pl.kernel,
    out_type=jax.ShapeDtypeStruct((E, DE), F32),
    mesh=_MESH,
    scratch_types=[
        pltpu.VMEM((EPW,), jnp.int32),
        pltpu.VMEM((EPW,), jnp.int32),
        pltpu.VMEM((NBUF, C, DE), F32),
        pltpu.VMEM((NBUF, C, DE), F32),
        pltpu.VMEM((NBUF, C, DE), F32),
        pltpu.SemaphoreType.DMA((NBUF,)),
        pltpu.SemaphoreType.DMA((NBUF,)),
        pltpu.SemaphoreType.DMA((NBUF,)),
    ],
    compiler_params=pltpu.CompilerParams(use_tc_tiling_on_sc=False),
)
def _sc_gather(p_hbm, q_hbm, src_hbm, dst_hbm, g_hbm,
               idx_s, idx_d, bufp, bufq, gout, semp, semq, semw):
    cid = lax.axis_index("c")
    sid = lax.axis_index("s")
    w = sid * NC + cid
    base = w * EPW
    pltpu.sync_copy(src_hbm.at[pl.ds(base, EPW)], idx_s)
    pltpu.sync_copy(dst_hbm.at[pl.ds(base, EPW)], idx_d)

    def gathers(k, ph):
        cp = pltpu.async_copy(p_hbm.at[idx_s.at[pl.ds(k * C, C)]],
                              bufp.at[ph], semp.at[ph])
        cq = pltpu.async_copy(q_hbm.at[idx_d.at[pl.ds(k * C, C)]],
                              bufq.at[ph], semq.at[ph])
        return cp, cq

    for ph in range(NBUF):
        gathers(ph, ph)

    def body(j, carry):
        for ph in range(NBUF):
            k = j * NBUF + ph
            pltpu.make_async_copy(p_hbm.at[idx_s.at[pl.ds(0, C)]],
                                  bufp.at[ph], semp.at[ph]).wait()
            pltpu.make_async_copy(q_hbm.at[idx_d.at[pl.ds(0, C)]],
                                  bufq.at[ph], semq.at[ph]).wait()
            @pl.when(j > 0)
            def _():
                pltpu.make_async_copy(gout.at[ph], g_hbm.at[pl.ds(0, C)],
                                      semw.at[ph]).wait()
            def addrow(r, c2):
                for cc in range(DE // 16):
                    gout[ph, r, pl.ds(cc * 16, 16)] = (
                        bufp[ph, r, pl.ds(cc * 16, 16)]
                        + bufq[ph, r, pl.ds(cc * 16, 16)])
                return c2
            lax.fori_loop(0, C, addrow, 0)
            pltpu.async_copy(gout.at[ph], g_hbm.at[pl.ds(base + k * C, C)],
                             semw.at[ph])
            kn = jnp.minimum(k + NBUF, KW - 1)
            @pl.when(k + NBUF < KW)
            def _():
                gathers(kn, ph)
        return carry

    lax.fori_loop(0, KSTEP, body, 0)
    for ph in range(NBUF):
        pltpu.make_async_copy(gout.at[ph], g_hbm.at[pl.ds(0, C)],
                              semw.at[ph]).wait()


EB = 16000
PK = 8
EBP = EB // PK

def _tcc_body(g_ref, hef8_ref, ef8_ref, w1bd, w2bd, whu8, wu8, hu_ref, u_ref,
              b8_ref, o32_ref, o16_ref):
    ec8 = (jnp.dot(hu_ref[...], whu8[...], preferred_element_type=F32)
           + jnp.dot(u_ref[...], wu8[...], preferred_element_type=F32)
           + b8_ref[...])
    g8 = jnp.reshape(g_ref[...], (EBP, PK * DE))
    s = (g8
         + jnp.dot(hef8_ref[...], w1bd[...], preferred_element_type=F32)
         + jnp.dot(ef8_ref[...], w2bd[...], preferred_element_type=F32)
         + ec8)
    t = jnp.tanh(s)
    h32 = jnp.concatenate([t[:, DE * j:DE * j + H] for j in range(PK)], axis=1)
    o32_ref[...] = jnp.reshape(h32, (EB * H // 128, 128))
    h16 = jnp.concatenate([t[:, DE * j + H:DE * (j + 1)] for j in range(PK)],
                          axis=1)
    o16_ref[...] = jnp.reshape(h16, (EB * DEDGE // 128, 128))


FZ = RZ // 2

@functools.partial(
    pl.kernel,
    out_type=jax.ShapeDtypeStruct((NC, NPAD, 128), F32),
    mesh=_MESH,
    scratch_types=[
        pltpu.VMEM((KW, C), jnp.int32),
        pltpu.VMEM((NBUF, C, H), F32),
        pltpu.VMEM((NBUF, C, DEDGE), F32),
        pltpu.VMEM_SHARED((NPAD, H), F32),
        pltpu.VMEM_SHARED((NPAD, DEDGE), F32),
        pltpu.VMEM((FZ, H), F32),
        pltpu.VMEM((FZ, DEDGE), F32),
        pltpu.VMEM((FZ, 128), F32),
        pltpu.SemaphoreType.DMA((NBUF,)),
        pltpu.SemaphoreType.DMA((NBUF,)),
        pltpu.SemaphoreType.DMA,
        pltpu.SemaphoreType.DMA,
    ],
    compiler_params=pltpu.CompilerParams(use_tc_tiling_on_sc=False),
)
def _sc_scatter(h32_hbm, h16_hbm, dst3_hbm, a48_hbm,
                idx2, buf32, buf16, sh32, sh16, f32b, f16b, mout,
                sem32, sem16, ssc32, ssc16):
    cid = lax.axis_index("c")
    sid = lax.axis_index("s")
    w = sid * NC + cid
    base = w * EPW
    zv = jnp.zeros((16,), F32)

    def zrow(r, carry):
        f32b[r, pl.ds(0, 16)] = zv
        f32b[r, pl.ds(16, 16)] = zv
        f16b[r, pl.ds(0, 16)] = zv
        return carry

    lax.fori_loop(0, FZ, zrow, 0)
    for m in range(RZ // FZ):
        pltpu.sync_copy(f32b, sh32.at[pl.ds(sid * RZ + m * FZ, FZ)])
        pltpu.sync_copy(f16b, sh16.at[pl.ds(sid * RZ + m * FZ, FZ)])
    pltpu.sync_copy(dst3_hbm.at[w], idx2)
    plsc.subcore_barrier()

    def loads(k, ph):
        off = base + k * C
        pltpu.async_copy(h32_hbm.at[pl.ds(off, C)], buf32.at[ph], sem32.at[ph])
        pltpu.async_copy(h16_hbm.at[pl.ds(off, C)], buf16.at[ph], sem16.at[ph])

    for ph in range(NBUF):
        loads(ph, ph)

    def body(j, carry):
        for ph in range(NBUF):
            k = j * NBUF + ph
            pltpu.make_async_copy(h32_hbm.at[pl.ds(0, C)], buf32.at[ph],
                                  sem32.at[ph]).wait()
            pltpu.make_async_copy(h16_hbm.at[pl.ds(0, C)], buf16.at[ph],
                                  sem16.at[ph]).wait()
            s32 = pltpu.async_copy(buf32.at[ph], sh32.at[idx2.at[k]], ssc32,
                                   add=True)
            s16 = pltpu.async_copy(buf16.at[ph], sh16.at[idx2.at[k]], ssc16,
                                   add=True)
            s32.wait()
            s16.wait()
            kn = jnp.minimum(k + NBUF, KW - 1)
            @pl.when(k + NBUF < KW)
            def _():
                loads(kn, ph)
        return carry

    lax.fori_loop(0, KSTEP, body, 0)
    plsc.subcore_barrier()
    for m in range(RZ // FZ):
        row0 = sid * RZ + m * FZ
        pltpu.sync_copy(sh32.at[pl.ds(row0, FZ)], f32b)
        pltpu.sync_copy(sh16.at[pl.ds(row0, FZ)], f16b)

        def mrow(r, carry):
            mout[r, pl.ds(0, 16)] = f32b[r, pl.ds(0, 16)]
            mout[r, pl.ds(16, 16)] = f32b[r, pl.ds(16, 16)]
            mout[r, pl.ds(32, 16)] = f16b[r, pl.ds(0, 16)]
            return carry

        lax.fori_loop(0, FZ, mrow, 0)
        pltpu.sync_copy(mout, a48_hbm.at[cid, pl.ds(row0, FZ)])


NB = 2000
NGRID = N // NB

def _tce_body(hnf_ref, nf_ref, a48_ref, wnh, wnn, wna32, wna16,
              wnhu, wnu, wnb, wghu, wgu, wgn, wge, wgb, hu_ref, u_ref,
              hnfn_ref, unfo_ref, hun_ref, uuo_ref, accn, acce):
    i = pl.program_id(0)
    a48 = a48_ref[0] + a48_ref[1]
    a32 = a48[:, :H]
    a16 = a48[:, H:DE]
    nc_term = (jnp.dot(hu_ref[...], wnhu[...], preferred_element_type=F32)
               + jnp.dot(u_ref[...], wnu[...], preferred_element_type=F32)
               + wnb[...])
    pre = (jnp.dot(hnf_ref[...], wnh[...], preferred_element_type=F32)
           + jnp.dot(nf_ref[...], wnn[...], preferred_element_type=F32)
           + jnp.dot(a32, wna32[...], preferred_element_type=F32)
           + jnp.dot(a16, wna16[...], preferred_element_type=F32)
           + nc_term)
    unf = jnp.tanh(pre)
    hnfn_ref[...] = unf[:, :H]
    unfo_ref[...] = unf[:, H:]

    @pl.when(i == 0)
    def _():
        accn[...] = jnp.zeros_like(accn)
        acce[...] = jnp.zeros_like(acce)

    accn[...] += jnp.sum(unf, axis=0, keepdims=True)
    acce[...] += jnp.concatenate(
        [jnp.sum(a32, axis=0, keepdims=True),
         jnp.sum(a16, axis=0, keepdims=True)], axis=1)

    ug = (jnp.dot(hu_ref[...], wghu[...], preferred_element_type=F32)
          + jnp.dot(u_ref[...], wgu[...], preferred_element_type=F32)
          + wgb[...])
    uu = jnp.tanh(ug + jnp.dot(accn[...], wgn[...], preferred_element_type=F32)
                  + jnp.dot(acce[...], wge[...], preferred_element_type=F32))
    hun_ref[...] = uu[:, :H]
    uuo_ref[...] = uu[:, H:]


def kernel(nf, ef, u, edge_index, hnf, hef, hu, We_w, We_b, Wn_w, Wn_b, Wg_w, Wg_b):
    src2 = edge_index[0]
    dst2 = edge_index[1]

    p, q = pl.pallas_call(
        _tca_body,
        out_shape=(jax.ShapeDtypeStruct((N, DE), F32),
                   jax.ShapeDtypeStruct((N, DE), F32)),
    )(hnf, nf, We_w[0:H], We_w[H:DN], We_w[DN:DN + H], We_w[DN + H:2 * DN])

    g = _sc_gather(p, q, src2, dst2)

    import jax.scipy.linalg as _jsl
    w1 = We_w[2 * DN:2 * DN + H]
    w2 = We_w[2 * DN + H:2 * DN + DE]
    w1bd = _jsl.block_diag(*([w1] * PK)).astype(jnp.bfloat16)
    w2bd = _jsl.block_diag(*([w2] * PK)).astype(jnp.bfloat16)
    whu8 = jnp.tile(We_w[2 * DN + DE:2 * DN + DE + H], (1, PK))
    wu8 = jnp.tile(We_w[2 * DN + DE + H:], (1, PK))
    b8 = jnp.tile(We_b.reshape(1, DE), (1, PK))
    g2 = g.reshape(E * DE // 128, 128)
    hef8 = hef.astype(jnp.bfloat16).reshape(E // PK, PK * H)
    ef8 = ef.astype(jnp.bfloat16).reshape(E // PK, PK * DEDGE)
    grid_e = E // EB
    h32p, h16p = pl.pallas_call(
        _tcc_body,
        grid=(grid_e,),
        in_specs=[
            pl.BlockSpec((EB * DE // 128, 128), lambda i: (i, 0)),
            pl.BlockSpec((EBP, PK * H), lambda i: (i, 0)),
            pl.BlockSpec((EBP, PK * DEDGE), lambda i: (i, 0)),
            pl.BlockSpec((PK * H, PK * DE), lambda i: (0, 0)),
            pl.BlockSpec((PK * DEDGE, PK * DE), lambda i: (0, 0)),
            pl.BlockSpec((H, PK * DE), lambda i: (0, 0)),
            pl.BlockSpec((DGLOB, PK * DE), lambda i: (0, 0)),
            pl.BlockSpec((1, H), lambda i: (0, 0)),
            pl.BlockSpec((1, DGLOB), lambda i: (0, 0)),
            pl.BlockSpec((1, PK * DE), lambda i: (0, 0)),
        ],
        out_specs=(pl.BlockSpec((EB * H // 128, 128), lambda i: (i, 0)),
                   pl.BlockSpec((EB * DEDGE // 128, 128), lambda i: (i, 0))),
        out_shape=(jax.ShapeDtypeStruct((E * H // 128, 128), F32),
                   jax.ShapeDtypeStruct((E * DEDGE // 128, 128), F32)),
        compiler_params=pltpu.CompilerParams(
            dimension_semantics=("parallel",)),
    )(g2, hef8, ef8, w1bd, w2bd, whu8, wu8, hu, u, b8)
    h32 = h32p.reshape(E, H)
    h16 = h16p.reshape(E, DEDGE)

    dst3 = dst2.reshape(NW, KW, C)
    a48 = _sc_scatter(h32, h16, dst3)

    hnfn, unfo, hun, uuo = pl.pallas_call(
        _tce_body,
        grid=(NGRID,),
        in_specs=[
            pl.BlockSpec((NB, H), lambda i: (i, 0)),
            pl.BlockSpec((NB, DFEAT), lambda i: (i, 0)),
            pl.BlockSpec((NC, NB, 128), lambda i: (0, i, 0)),
            pl.BlockSpec((H, DN), lambda i: (0, 0)),
            pl.BlockSpec((DFEAT, DN), lambda i: (0, 0)),
            pl.BlockSpec((H, DN), lambda i: (0, 0)),
            pl.BlockSpec((DEDGE, DN), lambda i: (0, 0)),
            pl.BlockSpec((H, DN), lambda i: (0, 0)),
            pl.BlockSpec((DGLOB, DN), lambda i: (0, 0)),
            pl.BlockSpec((1, DN), lambda i: (0, 0)),
            pl.BlockSpec((H, DU), lambda i: (0, 0)),
            pl.BlockSpec((DGLOB, DU), lambda i: (0, 0)),
            pl.BlockSpec((DN, DU), lambda i: (0, 0)),
            pl.BlockSpec((DE, DU), lambda i: (0, 0)),
            pl.BlockSpec((1, DU), lambda i: (0, 0)),
            pl.BlockSpec((1, H), lambda i: (0, 0)),
            pl.BlockSpec((1, DGLOB), lambda i: (0, 0)),
        ],
        out_specs=(pl.BlockSpec((NB, H), lambda i: (i, 0)),
                   pl.BlockSpec((NB, DFEAT), lambda i: (i, 0)),
                   pl.BlockSpec((1, H), lambda i: (0, 0)),
                   pl.BlockSpec((1, H), lambda i: (0, 0))),
        out_shape=(jax.ShapeDtypeStruct((N, H), F32),
                   jax.ShapeDtypeStruct((N, DFEAT), F32),
                   jax.ShapeDtypeStruct((1, H), F32),
                   jax.ShapeDtypeStruct((1, H), F32)),
        scratch_shapes=[pltpu.VMEM((1, DN), F32), pltpu.VMEM((1, DE), F32)],
        compiler_params=pltpu.CompilerParams(
            dimension_semantics=("arbitrary",)),
    )(hnf, nf, a48,
      Wn_w[0:H], Wn_w[H:DN], Wn_w[DN:DN + H], Wn_w[DN + H:DN + DE],
      Wn_w[DN + DE:DN + DE + H], Wn_w[DN + DE + H:], Wn_b.reshape(1, DN),
      Wg_w[0:H], Wg_w[H:DU], Wg_w[DU:DU + DN], Wg_w[DU + DN:],
      Wg_b.reshape(1, DU), hu, u)

    return (hnfn, h32, hun, unfo, h16, uuo)
